# speedup vs baseline: 1.4856x; 1.4856x over previous
import functools
import jax
import jax.numpy as jnp
from jax import lax
from jax.experimental import pallas as pl
from jax.experimental.pallas import tpu as pltpu
from jax.experimental.pallas import tpu_sc as plsc

D = 2048
L = 16
SCALE = float(D) ** 0.5

_info = plsc.get_sparse_core_info()
NC = _info.num_cores
NS = _info.num_subcores
NW = NC * NS

B = 16384
BPW = B // NW
C = 16
NCHUNK = BPW // C
NPAIR = NCHUNK // 2

_mesh = plsc.VectorSubcoreMesh(core_axis_name="c", subcore_axis_name="s")


@functools.partial(
    pl.kernel,
    mesh=_mesh,
    out_type=jax.ShapeDtypeStruct((B, D), jnp.float32),
    scratch_types=[
        pltpu.VMEM((BPW,), jnp.int32),
        pltpu.VMEM((2, C, D), jnp.float32),
        pltpu.SemaphoreType.DMA,
        pltpu.SemaphoreType.DMA,
        pltpu.SemaphoreType.DMA,
        pltpu.SemaphoreType.DMA,
    ],
)
def _embed(idx_hbm, table_hbm, out_hbm, idx_v, rows_v, g0, g1, s0, s1):
    wid = lax.axis_index("s") * NC + lax.axis_index("c")
    base = wid * BPW
    pltpu.sync_copy(idx_hbm.at[pl.ds(base, BPW)], idx_v)

    def g_copy(ci, buf, sem):
        return pltpu.make_async_copy(
            table_hbm.at[idx_v.at[pl.ds(ci * C, C)]],
            rows_v.at[buf],
            sem,
        )

    def s_copy(ci, buf, sem):
        return pltpu.make_async_copy(
            rows_v.at[buf],
            out_hbm.at[pl.ds(base + ci * C, C)],
            sem,
        )

    def scale(buf):
        def row_body(r, carry):
            for c in range(D // L):
                sl = pl.ds(c * L, L)
                rows_v[buf, r, sl] = rows_v[buf, r, sl] * SCALE
            return carry
        lax.fori_loop(0, C, row_body, 0)

    g_copy(0, 0, g0).start()

    def pair_body(p, carry):
        g = p * 2
        g_copy(g + 1, 1, g1).start()

        g_copy(g, 0, g0).wait()
        scale(0)
        s_copy(g, 0, s0).start()

        @pl.when(p < NPAIR - 1)
        def _start_next_even():
            s_copy(g, 0, s0).wait()
            g_copy(g + 2, 0, g0).start()

        g_copy(g + 1, 1, g1).wait()
        scale(1)
        s_copy(g + 1, 1, s1).start()

        @pl.when(p < NPAIR - 1)
        def _free_odd_buf():
            s_copy(g + 1, 1, s1).wait()

        return carry

    lax.fori_loop(0, NPAIR, pair_body, 0)

    s_copy(NCHUNK - 2, 0, s0).wait()
    s_copy(NCHUNK - 1, 1, s1).wait()


def kernel(input_ids, table):
    ids = input_ids.reshape(-1).astype(jnp.int32)
    out = _embed(ids, table)
    return out.reshape(input_ids.shape + (table.shape[1],))


# speedup vs baseline: 1.7041x; 1.1471x over previous
import functools
import jax
import jax.numpy as jnp
from jax import lax
from jax.experimental import pallas as pl
from jax.experimental.pallas import tpu as pltpu
from jax.experimental.pallas import tpu_sc as plsc

D = 2048
L = 16
SCALE = float(D) ** 0.5

_info = plsc.get_sparse_core_info()
NC = _info.num_cores
NS = _info.num_subcores
NW = NC * NS

B = 16384
BPW = B // NW
C = 8
NCHUNK = BPW // C
NBUF = 4
NOUTER = NCHUNK // NBUF

_mesh = plsc.VectorSubcoreMesh(core_axis_name="c", subcore_axis_name="s")


@functools.partial(
    pl.kernel,
    mesh=_mesh,
    out_type=jax.ShapeDtypeStruct((B, D), jnp.float32),
    scratch_types=[
        pltpu.VMEM((BPW,), jnp.int32),
        pltpu.VMEM((NBUF, C, D), jnp.float32),
    ]
    + [pltpu.SemaphoreType.DMA] * (2 * NBUF),
)
def _embed(idx_hbm, table_hbm, out_hbm, idx_v, rows_v, *sems):
    gsem = sems[:NBUF]
    ssem = sems[NBUF:]
    wid = lax.axis_index("s") * NC + lax.axis_index("c")
    base = wid * BPW
    pltpu.sync_copy(idx_hbm.at[pl.ds(base, BPW)], idx_v)

    def g_copy(ci, buf):
        return pltpu.make_async_copy(
            table_hbm.at[idx_v.at[pl.ds(ci * C, C)]],
            rows_v.at[buf],
            gsem[buf],
        )

    def s_copy(ci, buf):
        return pltpu.make_async_copy(
            rows_v.at[buf],
            out_hbm.at[pl.ds(base + ci * C, C)],
            ssem[buf],
        )

    def scale(buf):
        def row_body(r, carry):
            for c in range(D // L):
                sl = pl.ds(c * L, L)
                rows_v[buf, r, sl] = rows_v[buf, r, sl] * SCALE
            return carry
        lax.fori_loop(0, C, row_body, 0)

    for b in range(NBUF):
        g_copy(b, b).start()

    def outer_body(p, carry):
        for b in range(NBUF):
            ci = p * NBUF + b
            g_copy(ci, b).wait()
            scale(b)
            s_copy(ci, b).start()
            pb = (b - 2) % NBUF
            cj = ci + NBUF - 2

            @pl.when((ci >= 2) & (cj < NCHUNK))
            def _refill():
                s_copy(cj - NBUF, pb).wait()
                g_copy(cj, pb).start()

        return carry

    lax.fori_loop(0, NOUTER, outer_body, 0)

    for k in range(NBUF):
        ci = NCHUNK - NBUF + k
        s_copy(ci, ci % NBUF).wait()


def kernel(input_ids, table):
    ids = input_ids.reshape(-1).astype(jnp.int32)
    out = _embed(ids, table)
    return out.reshape(input_ids.shape + (table.shape[1],))


# speedup vs baseline: 1.7248x; 1.0122x over previous
import functools
import jax
import jax.numpy as jnp
from jax import lax
from jax.experimental import pallas as pl
from jax.experimental.pallas import tpu as pltpu
from jax.experimental.pallas import tpu_sc as plsc

D = 2048
L = 16
SCALE = float(D) ** 0.5

_info = plsc.get_sparse_core_info()
NC = _info.num_cores
NS = _info.num_subcores
NW = NC * NS

B = 16384
BPW = B // NW
C = 8
NCHUNK = BPW // C
NBUF = 4
DEFER = 1
NOUTER = NCHUNK // NBUF

_mesh = plsc.VectorSubcoreMesh(core_axis_name="c", subcore_axis_name="s")


@functools.partial(
    pl.kernel,
    mesh=_mesh,
    out_type=jax.ShapeDtypeStruct((B, D), jnp.float32),
    scratch_types=[
        pltpu.VMEM((BPW,), jnp.int32),
        pltpu.VMEM((NBUF, C, D), jnp.float32),
    ]
    + [pltpu.SemaphoreType.DMA] * (2 * NBUF),
)
def _embed(idx_hbm, table_hbm, out_hbm, idx_v, rows_v, *sems):
    gsem = sems[:NBUF]
    ssem = sems[NBUF:]
    wid = lax.axis_index("s") * NC + lax.axis_index("c")
    base = wid * BPW
    pltpu.sync_copy(idx_hbm.at[pl.ds(base, BPW)], idx_v)

    def g_copy(ci, buf):
        return pltpu.make_async_copy(
            table_hbm.at[idx_v.at[pl.ds(ci * C, C)]],
            rows_v.at[buf],
            gsem[buf],
        )

    def s_copy(ci, buf):
        return pltpu.make_async_copy(
            rows_v.at[buf],
            out_hbm.at[pl.ds(base + ci * C, C)],
            ssem[buf],
        )

    def scale(buf):
        def row_body(r, carry):
            for c in range(D // L):
                sl = pl.ds(c * L, L)
                rows_v[buf, r, sl] = rows_v[buf, r, sl] * SCALE
            return carry
        lax.fori_loop(0, C, row_body, 0)

    for b in range(NBUF):
        g_copy(b, b).start()

    def outer_body(p, carry):
        for b in range(NBUF):
            ci = p * NBUF + b
            g_copy(ci, b).wait()
            scale(b)
            s_copy(ci, b).start()
            pb = (b - DEFER) % NBUF
            cj = ci + NBUF - DEFER

            @pl.when((ci >= DEFER) & (cj < NCHUNK))
            def _refill():
                s_copy(cj - NBUF, pb).wait()
                g_copy(cj, pb).start()

        return carry

    lax.fori_loop(0, NOUTER, outer_body, 0)

    for k in range(NBUF):
        ci = NCHUNK - NBUF + k
        s_copy(ci, ci % NBUF).wait()


def kernel(input_ids, table):
    ids = input_ids.reshape(-1).astype(jnp.int32)
    out = _embed(ids, table)
    return out.reshape(input_ids.shape + (table.shape[1],))
